# Initial kernel scaffold; baseline (speedup 1.0000x reference)
import jax
import jax.numpy as jnp
from jax.experimental import pallas as pl


def kernel(K1_8, H1, M1):
    raise NotImplementedError("write your pallas kernel here")


# speedup vs baseline: 1.0129x; 1.0129x over previous
import functools

import jax
import jax.numpy as jnp
from jax.experimental import pallas as pl
from jax.experimental.pallas import tpu as pltpu

_CLS = 8
_TOPK = 4096
_THR = 0.05
_NEG_INF = float("-inf")


def _softmax_norm_body(k18_ref, m1_ref, probs_ref, m1n_ref):
    x = k18_ref[0]
    m = jnp.max(x, axis=0, keepdims=True)
    e = jnp.exp(x - m)
    s = jnp.sum(e, axis=0, keepdims=True)
    probs_ref[0] = e[:64] / s

    f = m1_ref[0]
    n = jnp.sqrt(jnp.sum(f * f, axis=0, keepdims=True))
    m1n_ref[0] = f / jnp.maximum(n, 1e-12)


def _softmax_norm(k18, m1):
    B = k18.shape[0]
    return pl.pallas_call(
        _softmax_norm_body,
        grid=(B,),
        in_specs=[
            pl.BlockSpec((1, 65, 4096), lambda b: (b, 0, 0)),
            pl.BlockSpec((1, 64, 4096), lambda b: (b, 0, 0)),
        ],
        out_specs=[
            pl.BlockSpec((1, 64, 4096), lambda b: (b, 0, 0)),
            pl.BlockSpec((1, 64, 4096), lambda b: (b, 0, 0)),
        ],
        out_shape=[
            jax.ShapeDtypeStruct((B, 64, 4096), jnp.float32),
            jax.ShapeDtypeStruct((B, 64, 4096), jnp.float32),
        ],
    )(k18, m1)


def _shift_max_rows(x, d):
    H = x.shape[0]
    pad = jnp.full((d, x.shape[1]), _NEG_INF, x.dtype)
    up = jnp.concatenate([x[d:], pad], axis=0)
    dn = jnp.concatenate([pad, x[:H - d]], axis=0)
    return jnp.maximum(up, dn)


def _shift_max_cols(x, d):
    W = x.shape[1]
    pad = jnp.full((x.shape[0], d), _NEG_INF, x.dtype)
    lf = jnp.concatenate([x[:, d:], pad], axis=1)
    rt = jnp.concatenate([pad, x[:, :W - d]], axis=1)
    return jnp.maximum(lf, rt)


def _nms_score_body(heat_ref, h1_ref, score_ref):
    h = heat_ref[0]
    rm = jnp.maximum(h, jnp.maximum(_shift_max_rows(h, 1), _shift_max_rows(h, 2)))
    cm = jnp.maximum(rm, jnp.maximum(_shift_max_cols(rm, 1), _shift_max_cols(rm, 2)))
    pos = (h == cm) & (h > _THR)
    score_ref[0] = jnp.where(pos, h * h1_ref[0], -1.0)


def _nms_score(heat, h1):
    B = heat.shape[0]
    return pl.pallas_call(
        _nms_score_body,
        grid=(B,),
        in_specs=[
            pl.BlockSpec((1, 512, 512), lambda b: (b, 0, 0)),
            pl.BlockSpec((1, 512, 512), lambda b: (b, 0, 0)),
        ],
        out_specs=pl.BlockSpec((1, 512, 512), lambda b: (b, 0, 0)),
        out_shape=jax.ShapeDtypeStruct((B, 512, 512), jnp.float32),
    )(heat, h1)


def _bilinear(table, idx, Hh, Ww):
    xs = (idx % Ww).astype(jnp.float32)
    ys = (idx // Ww).astype(jnp.float32)
    gx = 2.0 * xs / (Ww - 1) - 1.0
    gy = 2.0 * ys / (Hh - 1) - 1.0
    ix = ((gx + 1.0) * 64 - 1.0) / 2.0
    iy = ((gy + 1.0) * 64 - 1.0) / 2.0
    x0 = jnp.floor(ix); y0 = jnp.floor(iy)
    wx1 = ix - x0; wx0 = 1.0 - wx1
    wy1 = iy - y0; wy0 = 1.0 - wy1

    def g(yy, xx):
        valid = (xx >= 0) & (xx <= 63) & (yy >= 0) & (yy <= 63)
        xc = jnp.clip(xx, 0, 63).astype(jnp.int32)
        yc = jnp.clip(yy, 0, 63).astype(jnp.int32)
        return table[yc * 64 + xc] * valid.astype(jnp.float32)[:, None]

    return (g(y0, x0) * (wy0 * wx0)[:, None] + g(y0, x0 + 1) * (wy0 * wx1)[:, None]
            + g(y0 + 1, x0) * (wy1 * wx0)[:, None] + g(y0 + 1, x0 + 1) * (wy1 * wx1)[:, None])


def kernel(K1_8, H1, M1):
    B = K1_8.shape[0]
    Hh, Ww = H1.shape[-2], H1.shape[-1]

    probs, m1n = _softmax_norm(K1_8.reshape(B, 65, 4096), M1.reshape(B, 64, 4096))
    heat = probs.reshape(B, 8, 8, 64, 64).transpose(0, 3, 1, 4, 2).reshape(B, 512, 512)
    table = m1n.reshape(B, 64, 4096).transpose(0, 2, 1)

    scores_dense = _nms_score(heat, H1.reshape(B, 512, 512)).reshape(B, -1)

    scores, idx = jax.lax.top_k(scores_dense, _TOPK)
    xs = (idx % Ww).astype(jnp.float32)
    ys = (idx // Ww).astype(jnp.float32)
    mkpts = jnp.stack([xs, ys], axis=-1)
    feats = jax.vmap(lambda t, i: _bilinear(t, i, Hh, Ww))(table, idx)
    feats = feats / jnp.maximum(jnp.linalg.norm(feats, axis=-1, keepdims=True), 1e-12)
    valid = scores > 0
    return scores, mkpts, feats, valid
